# baseline (speedup 1.0000x reference)
import jax
import jax.numpy as jnp
from jax.experimental import pallas as pl
from jax.experimental.pallas import tpu as pltpu

E = 64
D = 1024
DFF = 4096
BLK = 128
DFFT = 2048
K = DFF // DFFT


def _ffn_seg_kernel(seg_e, seg_k, seg_b, seg_r0, seg_r1,
                    x_ref, cent_ref, lns_ref, lnb_ref,
                    w1_ref, b1_ref, w2a_ref, w2b_ref, b2_ref,
                    out_ref, acc_scr):
    t = pl.program_id(0)
    k = seg_k[t]
    row0 = seg_b[t] * BLK

    x = x_ref[...]
    mu = jnp.mean(x, axis=1, keepdims=True)
    var = jnp.mean((x - mu) * (x - mu), axis=1, keepdims=True)
    xn = (x - mu) * jax.lax.rsqrt(var + 1e-5)
    xn = xn * lns_ref[0, 0][None, :] + lnb_ref[0, 0][None, :]
    xn = xn.astype(jnp.bfloat16)

    w1 = w1_ref[0].astype(jnp.bfloat16)
    b1 = b1_ref[0, 0]
    h = jax.lax.dot_general(xn, w1, (((1,), (1,)), ((), ())),
                            preferred_element_type=jnp.float32)
    h = jnp.maximum(h + b1[None, :], 0.0).astype(jnp.bfloat16)
    w2a = w2a_ref[0, 0].astype(jnp.bfloat16)
    w2b = w2b_ref[0, 0].astype(jnp.bfloat16)
    part_a = jax.lax.dot_general(h, w2a, (((1,), (1,)), ((), ())),
                                 preferred_element_type=jnp.float32)
    part_b = jax.lax.dot_general(h, w2b, (((1,), (1,)), ((), ())),
                                 preferred_element_type=jnp.float32)

    @pl.when(k == 0)
    def _():
        acc_scr[pl.ds(row0, BLK), 0:D // 2] = part_a
        acc_scr[pl.ds(row0, BLK), D // 2:D] = part_b

    @pl.when(k != 0)
    def _():
        acc_scr[pl.ds(row0, BLK), 0:D // 2] += part_a
        acc_scr[pl.ds(row0, BLK), D // 2:D] += part_b

    @pl.when(k == K - 1)
    def _():
        c = cent_ref[0, 0]
        logit = jnp.sum(x * c[None, :], axis=1, keepdims=True)
        alpha = jax.nn.sigmoid(logit)
        y = x + alpha * (acc_scr[pl.ds(row0, BLK), :] + b2_ref[0, 0][None, :])
        rows = jax.lax.broadcasted_iota(jnp.int32, (BLK, 1), 0)
        mask = (rows >= seg_r0[t]) & (rows < seg_r1[t])
        out_ref[pl.ds(row0, BLK), :] = jnp.where(
            mask, y, out_ref[pl.ds(row0, BLK), :])


def _expert_ffn(routed, seg_e, seg_k, seg_b, seg_r0, seg_r1,
                expert_centroids, ln_scale, ln_bias, W1, b1, W2, b2):
    T = routed.shape[0]
    nstep = seg_e.shape[0]
    cent3 = expert_centroids.reshape(E, 1, D)
    lns3 = ln_scale.reshape(E, 1, D)
    lnb3 = ln_bias.reshape(E, 1, D)
    b1_3 = b1.reshape(E, 1, DFF)
    b2_3 = b2.reshape(E, 1, D)

    grid_spec = pltpu.PrefetchScalarGridSpec(
        num_scalar_prefetch=5,
        grid=(nstep,),
        in_specs=[
            pl.BlockSpec((BLK, D), lambda t, se, sk, sb, r0, r1: (sb[t], 0)),
            pl.BlockSpec((1, 1, D), lambda t, se, sk, sb, r0, r1: (se[t], 0, 0)),
            pl.BlockSpec((1, 1, D), lambda t, se, sk, sb, r0, r1: (se[t], 0, 0)),
            pl.BlockSpec((1, 1, D), lambda t, se, sk, sb, r0, r1: (se[t], 0, 0)),
            pl.BlockSpec((1, DFFT, D), lambda t, se, sk, sb, r0, r1: (se[t], sk[t], 0)),
            pl.BlockSpec((1, 1, DFFT), lambda t, se, sk, sb, r0, r1: (se[t], 0, sk[t])),
            pl.BlockSpec((1, 1, D // 2, DFFT),
                         lambda t, se, sk, sb, r0, r1: (se[t], 0, 0, sk[t])),
            pl.BlockSpec((1, 1, D // 2, DFFT),
                         lambda t, se, sk, sb, r0, r1: (se[t], 1, 0, sk[t])),
            pl.BlockSpec((1, 1, D), lambda t, se, sk, sb, r0, r1: (se[t], 0, 0)),
        ],
        out_specs=pl.BlockSpec((T, D), lambda t, se, sk, sb, r0, r1: (0, 0)),
        scratch_shapes=[pltpu.VMEM((T, D), jnp.float32)],
    )
    W2r = W2.reshape(E, 2, D // 2, DFF)
    return pl.pallas_call(
        _ffn_seg_kernel,
        grid_spec=grid_spec,
        out_shape=jax.ShapeDtypeStruct((T, D), jnp.float32),
        compiler_params=pltpu.CompilerParams(
            dimension_semantics=("arbitrary",),
            vmem_limit_bytes=64 * 1024 * 1024,
        ),
    )(seg_e, seg_k, seg_b, seg_r0, seg_r1,
      routed, cent3, lns3, lnb3, W1, b1_3, W2r, W2r, b2_3)


def kernel(input_features, expert_centroids, ln_scale, ln_bias, W1, b1, W2, b2):
    shape = input_features.shape
    x = input_features.reshape(-1, shape[-1])
    T = x.shape[0]
    nseg = (T // BLK) + E - 1
    nstep = K * nseg

    scores = x @ expert_centroids.T
    tok_e = jnp.argmax(scores, axis=1).astype(jnp.int32)
    order = jnp.argsort(tok_e).astype(jnp.int32)
    routed = x[order]

    counts = jnp.bincount(tok_e, length=E)
    off = jnp.concatenate([jnp.zeros((1,), jnp.int32),
                           jnp.cumsum(counts).astype(jnp.int32)])
    cnt = off[1:] - off[:-1]
    fb = off[:-1] // BLK
    lb = jnp.where(cnt > 0, (off[1:] - 1) // BLK, fb - 1)
    m = jnp.where(cnt > 0, lb - fb + 1, 0)
    cumf = jnp.cumsum(K * m).astype(jnp.int32)
    flat_start = jnp.concatenate([jnp.zeros((1,), jnp.int32), cumf])
    total = flat_start[E]

    t_idx = jnp.arange(nstep, dtype=jnp.int32)
    e_t = jnp.searchsorted(cumf, t_idx, side='right').astype(jnp.int32)
    e_t = jnp.minimum(e_t, E - 1)
    local = t_idx - flat_start[e_t]
    m_t = jnp.maximum(m[e_t], 1)
    k_t = local // m_t
    b_t = fb[e_t] + (local % m_t)
    r0 = jnp.maximum(off[e_t], b_t * BLK) - b_t * BLK
    r1 = jnp.minimum(off[e_t + 1], (b_t + 1) * BLK) - b_t * BLK
    valid = t_idx < total
    last = jnp.maximum(total - 1, 0)
    e_t = jnp.where(valid, e_t, e_t[last]).astype(jnp.int32)
    k_t = jnp.where(valid, k_t, K - 1).astype(jnp.int32)
    b_t = jnp.where(valid, b_t, b_t[last]).astype(jnp.int32)
    r0 = jnp.where(valid, r0, 0).astype(jnp.int32)
    r1 = jnp.where(valid, r1, 0).astype(jnp.int32)

    out_sorted = _expert_ffn(routed, e_t, k_t, b_t, r0, r1,
                             expert_centroids, ln_scale, ln_bias, W1, b1, W2, b2)

    inv = jnp.zeros((T,), jnp.int32).at[order].set(
        jnp.arange(T, dtype=jnp.int32))
    result = out_sorted[inv]
    return result.reshape(shape)
